# baseline (speedup 1.0000x reference)
import functools
import math

import jax
import jax.numpy as jnp
from jax import lax
from jax.experimental import pallas as pl
from jax.experimental.pallas import tpu as pltpu
from jax.experimental.pallas import tpu_sc as plsc

D_MODEL = 64
N_POS = 16384
N_J = 50
SCALE = math.sqrt(D_MODEL)

_info = plsc.get_sparse_core_info()
_NC, _NS = _info.num_cores, _info.num_subcores
_NW = _NC * _NS
CI = N_POS // _NW
NBUF = 3


def _gather_body(xt_hbm, lut_hbm, g_hbm, idx_v, rbufs, sems_g, sems_o):
    wid = lax.axis_index("s") * _NC + lax.axis_index("c")
    pltpu.sync_copy(xt_hbm.at[:, pl.ds(wid * CI, CI)], idx_v)

    def gather_start(j, b):
        for k in range(CI // 128):
            pltpu.async_copy(
                lut_hbm.at[idx_v.at[j, pl.ds(k * 128, 128)]],
                rbufs[b].at[pl.ds(k * 128, 128)],
                sems_g[b],
            )

    def gather_wait(j, b):
        for k in range(CI // 128):
            pltpu.make_async_copy(
                lut_hbm.at[idx_v.at[j, pl.ds(k * 128, 128)]],
                rbufs[b].at[pl.ds(k * 128, 128)],
                sems_g[b],
            ).wait()

    def out_start(j, b):
        pltpu.async_copy(rbufs[b], g_hbm.at[wid, j], sems_o[b])

    def out_wait(j, b):
        pltpu.make_async_copy(rbufs[b], g_hbm.at[wid, j], sems_o[b]).wait()

    def step(j, b, with_out_wait, with_gather):
        gather_wait(j, b)
        if with_out_wait:
            out_wait(j - 1, (b + 2) % NBUF)
        out_start(j, b)
        if with_gather:
            gather_start(j + 2, (b + 2) % NBUF)

    gather_start(0, 0)
    gather_start(1, 1)
    for j in range(3):
        step(j, j % NBUF, j >= 1, True)

    def inner(tt, c2):
        for b in range(NBUF):
            j = NBUF * tt + b
            step(j, b, True, True)
        return c2

    lax.fori_loop(1, 16, inner, 0)

    for j in range(48, N_J):
        step(j, j % NBUF, True, False)
    out_wait(N_J - 1, (N_J - 1) % NBUF)


_gather = functools.partial(
    pl.kernel,
    out_type=jax.ShapeDtypeStruct((_NW, N_J, CI, D_MODEL), jnp.float32),
    mesh=plsc.VectorSubcoreMesh(core_axis_name="c", subcore_axis_name="s"),
    scratch_types=[
        pltpu.VMEM((N_J, CI), jnp.int32),
        [pltpu.VMEM((CI, D_MODEL), jnp.float32) for _ in range(NBUF)],
        [pltpu.SemaphoreType.DMA for _ in range(NBUF)],
        [pltpu.SemaphoreType.DMA for _ in range(NBUF)],
    ],
    compiler_params=pltpu.CompilerParams(
        use_tc_tiling_on_sc=False, needs_layout_passes=False
    ),
)(_gather_body)


def _tr_body(g_ref, o_ref):
    o_ref[0] = jnp.swapaxes(g_ref[0, 0], 0, 1) * SCALE


_transpose = pl.pallas_call(
    _tr_body,
    grid=(N_J, _NW),
    in_specs=[
        pl.BlockSpec((1, 1, CI, D_MODEL), lambda j, w: (w, j, 0, 0)),
    ],
    out_specs=pl.BlockSpec((1, D_MODEL, CI), lambda j, w: (j, 0, w)),
    out_shape=jax.ShapeDtypeStruct((N_J, D_MODEL, N_POS), jnp.float32),
)


@jax.jit
def kernel(x, lut):
    xt = x.T.astype(jnp.int32)
    g = _gather(xt, lut)
    outp = _transpose(g)
    return outp.transpose(2, 0, 1)


# speedup vs baseline: 1.5169x; 1.5169x over previous
import functools
import math

import jax
import jax.numpy as jnp
from jax import lax
from jax.experimental import pallas as pl
from jax.experimental.pallas import tpu as pltpu
from jax.experimental.pallas import tpu_sc as plsc

D_MODEL = 64
N_POS = 16384
N_J = 50
SCALE = math.sqrt(D_MODEL)

_info = plsc.get_sparse_core_info()
_NC, _NS = _info.num_cores, _info.num_subcores
_NW = _NC * _NS
CI = N_POS // _NW
NBUF = 3


def _gather_body(xt_hbm, lut_hbm, g_hbm, idx_v, rbufs, sems_g, sems_o):
    wid = lax.axis_index("s") * _NC + lax.axis_index("c")
    pltpu.sync_copy(xt_hbm.at[:, pl.ds(wid * CI, CI)], idx_v)

    def gather_start(j, b):
        for k in range(CI // 128):
            pltpu.async_copy(
                lut_hbm.at[idx_v.at[j, pl.ds(k * 128, 128)]],
                rbufs[b].at[pl.ds(k * 128, 128)],
                sems_g[b],
            )

    def gather_wait(j, b):
        for k in range(CI // 128):
            pltpu.make_async_copy(
                lut_hbm.at[idx_v.at[j, pl.ds(k * 128, 128)]],
                rbufs[b].at[pl.ds(k * 128, 128)],
                sems_g[b],
            ).wait()

    def out_start(j, b):
        pltpu.async_copy(rbufs[b], g_hbm.at[wid, j], sems_o[b])

    def out_wait(j, b):
        pltpu.make_async_copy(rbufs[b], g_hbm.at[wid, j], sems_o[b]).wait()

    def step(j, b, with_out_wait, with_gather):
        gather_wait(j, b)
        if with_out_wait:
            out_wait(j - 1, (b + 2) % NBUF)
        out_start(j, b)
        if with_gather:
            gather_start(j + 2, (b + 2) % NBUF)

    gather_start(0, 0)
    gather_start(1, 1)
    for j in range(3):
        step(j, j % NBUF, j >= 1, True)

    def inner(tt, c2):
        for b in range(NBUF):
            j = NBUF * tt + b
            step(j, b, True, True)
        return c2

    lax.fori_loop(1, 16, inner, 0)

    for j in range(48, N_J):
        step(j, j % NBUF, True, False)
    out_wait(N_J - 1, (N_J - 1) % NBUF)


_gather = functools.partial(
    pl.kernel,
    out_type=jax.ShapeDtypeStruct((_NW, N_J, CI, D_MODEL), jnp.float32),
    mesh=plsc.VectorSubcoreMesh(core_axis_name="c", subcore_axis_name="s"),
    scratch_types=[
        pltpu.VMEM((N_J, CI), jnp.int32),
        [pltpu.VMEM((CI, D_MODEL), jnp.float32) for _ in range(NBUF)],
        [pltpu.SemaphoreType.DMA for _ in range(NBUF)],
        [pltpu.SemaphoreType.DMA for _ in range(NBUF)],
    ],
    compiler_params=pltpu.CompilerParams(
        use_tc_tiling_on_sc=False, needs_layout_passes=False
    ),
)(_gather_body)


_WB = 8


def _tr_body(g_ref, o_ref):
    for ww in range(_WB):
        o_ref[0, :, ww * CI : (ww + 1) * CI] = (
            jnp.swapaxes(g_ref[ww, 0], 0, 1) * SCALE
        )


_transpose = pl.pallas_call(
    _tr_body,
    grid=(N_J, _NW // _WB),
    in_specs=[
        pl.BlockSpec((_WB, 1, CI, D_MODEL), lambda j, wb: (wb, j, 0, 0)),
    ],
    out_specs=pl.BlockSpec((1, D_MODEL, _WB * CI), lambda j, wb: (j, 0, wb)),
    out_shape=jax.ShapeDtypeStruct((N_J, D_MODEL, N_POS), jnp.float32),
)


@jax.jit
def kernel(x, lut):
    xt = x.T.astype(jnp.int32)
    g = _gather(xt, lut)
    outp = _transpose(g)
    return outp.transpose(2, 0, 1)


# speedup vs baseline: 1.5994x; 1.0544x over previous
import functools
import math

import jax
import jax.numpy as jnp
from jax import lax
from jax.experimental import pallas as pl
from jax.experimental.pallas import tpu as pltpu
from jax.experimental.pallas import tpu_sc as plsc

D_MODEL = 64
N_POS = 16384
N_J = 50
SCALE = math.sqrt(D_MODEL)

_info = plsc.get_sparse_core_info()
_NC, _NS = _info.num_cores, _info.num_subcores
_NW = _NC * _NS
CI = N_POS // _NW
NBUF = 3


def _gather_body(xt_hbm, lut_hbm, g_hbm, idx_v, rbufs, sems_g, sems_o):
    wid = lax.axis_index("s") * _NC + lax.axis_index("c")
    pltpu.sync_copy(xt_hbm.at[:, pl.ds(wid * CI, CI)], idx_v)

    def gather_start(j, b):
        for k in range(CI // 128):
            pltpu.async_copy(
                lut_hbm.at[idx_v.at[j, pl.ds(k * 128, 128)]],
                rbufs[b].at[pl.ds(k * 128, 128)],
                sems_g[b],
            )

    def gather_wait(j, b):
        for k in range(CI // 128):
            pltpu.make_async_copy(
                lut_hbm.at[idx_v.at[j, pl.ds(k * 128, 128)]],
                rbufs[b].at[pl.ds(k * 128, 128)],
                sems_g[b],
            ).wait()

    def out_start(j, b):
        pltpu.async_copy(rbufs[b], g_hbm.at[wid, j], sems_o[b])

    def out_wait(j, b):
        pltpu.make_async_copy(rbufs[b], g_hbm.at[wid, j], sems_o[b]).wait()

    def step(j, b, with_out_wait, with_gather):
        gather_wait(j, b)
        if with_out_wait:
            out_wait(j - 1, (b + 2) % NBUF)
        out_start(j, b)
        if with_gather:
            gather_start(j + 2, (b + 2) % NBUF)

    gather_start(0, 0)
    gather_start(1, 1)
    for j in range(3):
        step(j, j % NBUF, j >= 1, True)

    def inner(tt, c2):
        for b in range(NBUF):
            j = NBUF * tt + b
            step(j, b, True, True)
        return c2

    lax.fori_loop(1, 16, inner, 0)

    for j in range(48, N_J):
        step(j, j % NBUF, True, False)
    out_wait(N_J - 1, (N_J - 1) % NBUF)


_gather = functools.partial(
    pl.kernel,
    out_type=jax.ShapeDtypeStruct((_NW, N_J, CI, D_MODEL), jnp.float32),
    mesh=plsc.VectorSubcoreMesh(core_axis_name="c", subcore_axis_name="s"),
    scratch_types=[
        pltpu.VMEM((N_J, CI), jnp.int32),
        [pltpu.VMEM((CI, D_MODEL), jnp.float32) for _ in range(NBUF)],
        [pltpu.SemaphoreType.DMA for _ in range(NBUF)],
        [pltpu.SemaphoreType.DMA for _ in range(NBUF)],
    ],
    compiler_params=pltpu.CompilerParams(
        use_tc_tiling_on_sc=False, needs_layout_passes=False
    ),
)(_gather_body)


_WB = 32


def _tr_body(g_ref, o_ref):
    for ww in range(_WB):
        o_ref[0, :, ww * CI : (ww + 1) * CI] = (
            jnp.swapaxes(g_ref[ww, 0], 0, 1) * SCALE
        )


_transpose = pl.pallas_call(
    _tr_body,
    grid=(N_J, _NW // _WB),
    in_specs=[
        pl.BlockSpec((_WB, 1, CI, D_MODEL), lambda j, wb: (wb, j, 0, 0)),
    ],
    out_specs=pl.BlockSpec((1, D_MODEL, _WB * CI), lambda j, wb: (j, 0, wb)),
    out_shape=jax.ShapeDtypeStruct((N_J, D_MODEL, N_POS), jnp.float32),
)


@jax.jit
def kernel(x, lut):
    xt = x.T.astype(jnp.int32)
    g = _gather(xt, lut)
    outp = _transpose(g)
    return outp.transpose(2, 0, 1)


# speedup vs baseline: 1.6270x; 1.0172x over previous
import functools
import math

import jax
import jax.numpy as jnp
from jax import lax
from jax.experimental import pallas as pl
from jax.experimental.pallas import tpu as pltpu
from jax.experimental.pallas import tpu_sc as plsc

D_MODEL = 64
N_POS = 16384
N_J = 50
N_JH = N_J // 2
SCALE = math.sqrt(D_MODEL)

_info = plsc.get_sparse_core_info()
_NC, _NS = _info.num_cores, _info.num_subcores
_NW = _NC * _NS
CI = N_POS // _NW
NBUF = 3


def _gather_body(xt_hbm, lut_hbm, g_hbm, idx_v, rbufs, sems_g, sems_o):
    wid = lax.axis_index("s") * _NC + lax.axis_index("c")
    pltpu.sync_copy(xt_hbm.at[:, pl.ds(wid * CI, CI)], idx_v)

    def gather_start(j, b):
        for k in range(CI // 128):
            pltpu.async_copy(
                lut_hbm.at[idx_v.at[j, pl.ds(k * 128, 128)]],
                rbufs[b].at[pl.ds(k * 128, 128)],
                sems_g[b],
            )

    def gather_wait(j, b):
        for k in range(CI // 128):
            pltpu.make_async_copy(
                lut_hbm.at[idx_v.at[j, pl.ds(k * 128, 128)]],
                rbufs[b].at[pl.ds(k * 128, 128)],
                sems_g[b],
            ).wait()

    def out_start(j, b):
        pltpu.async_copy(rbufs[b], g_hbm.at[wid, j], sems_o[b])

    def out_wait(j, b):
        pltpu.make_async_copy(rbufs[b], g_hbm.at[wid, j], sems_o[b]).wait()

    def step(j, b, with_out_wait, with_gather):
        gather_wait(j, b)
        if with_out_wait:
            out_wait(j - 1, (b + 2) % NBUF)
        out_start(j, b)
        if with_gather:
            gather_start(j + 2, (b + 2) % NBUF)

    gather_start(0, 0)
    gather_start(1, 1)
    for j in range(3):
        step(j, j % NBUF, j >= 1, True)

    def inner(tt, c2):
        for b in range(NBUF):
            j = NBUF * tt + b
            step(j, b, True, True)
        return c2

    lax.fori_loop(1, 7, inner, 0)

    for j in range(21, 23):
        step(j, j % NBUF, True, True)
    for j in range(23, N_JH):
        step(j, j % NBUF, True, False)
    out_wait(N_JH - 1, (N_JH - 1) % NBUF)


_gather = functools.partial(
    pl.kernel,
    out_type=jax.ShapeDtypeStruct((_NW, N_JH, CI, D_MODEL), jnp.float32),
    mesh=plsc.VectorSubcoreMesh(core_axis_name="c", subcore_axis_name="s"),
    scratch_types=[
        pltpu.VMEM((N_JH, CI), jnp.int32),
        [pltpu.VMEM((CI, D_MODEL), jnp.float32) for _ in range(NBUF)],
        [pltpu.SemaphoreType.DMA for _ in range(NBUF)],
        [pltpu.SemaphoreType.DMA for _ in range(NBUF)],
    ],
    compiler_params=pltpu.CompilerParams(
        use_tc_tiling_on_sc=False, needs_layout_passes=False
    ),
)(_gather_body)


def _tr_body_first(g_ref, o_ref):
    for ww in range(_NW):
        o_ref[0, :, ww * CI : (ww + 1) * CI] = (
            jnp.swapaxes(g_ref[ww, 0], 0, 1) * SCALE
        )


def _tr_body_second(g_ref, prev_ref, o_ref):
    del prev_ref
    _tr_body_first(g_ref, o_ref)


_OUT_SHAPE = jax.ShapeDtypeStruct((N_J, D_MODEL, N_POS), jnp.float32)
_G_SPEC = pl.BlockSpec((_NW, 1, CI, D_MODEL), lambda j: (0, j, 0, 0))

_tr_first = pl.pallas_call(
    _tr_body_first,
    grid=(N_JH,),
    in_specs=[_G_SPEC],
    out_specs=pl.BlockSpec((1, D_MODEL, N_POS), lambda j: (j, 0, 0)),
    out_shape=_OUT_SHAPE,
)

_tr_second = pl.pallas_call(
    _tr_body_second,
    grid=(N_JH,),
    in_specs=[_G_SPEC, pl.BlockSpec(memory_space=pltpu.MemorySpace.HBM)],
    out_specs=pl.BlockSpec((1, D_MODEL, N_POS), lambda j: (j + N_JH, 0, 0)),
    out_shape=_OUT_SHAPE,
    input_output_aliases={1: 0},
)


@jax.jit
def kernel(x, lut):
    xt = x.T.astype(jnp.int32)
    g_a = _gather(xt[:N_JH], lut)
    g_b = _gather(xt[N_JH:], lut)
    o1 = _tr_first(g_a)
    out = _tr_second(g_b, o1)
    return out.transpose(2, 0, 1)
